# baseline (speedup 1.0000x reference)
import functools

import jax
import jax.numpy as jnp
from jax import lax
from jax.experimental import pallas as pl
from jax.experimental.pallas import tpu as pltpu
from jax.experimental.pallas import tpu_sc as plsc

H = 768
L = 16
NC = 2
NS = 16
NW = NC * NS
HV = H // L
EPS = 1e-5
CHUNK = 32
INV_H = 1.0 / H


def _rsqrt_vec(x):
  i = lax.bitcast_convert_type(x, jnp.int32)
  i = jnp.int32(0x5F3759DF) - lax.shift_right_arithmetic(i, 1)
  y = lax.bitcast_convert_type(i, jnp.float32)
  half_x = x * jnp.float32(0.5)
  for _ in range(2):
    y = y * (jnp.float32(1.5) - half_x * y * y)
  return y


def _ln_chunk(inb, outb, gam):
  @plsc.parallel_loop(0, CHUNK, unroll=2)
  def row_body(r):
    acc = [jnp.zeros((L,), jnp.float32) for _ in range(8)]
    for j in range(HV):
      v = inb[r, pl.ds(j * L, L)]
      k = j % 4
      acc[k] = acc[k] + v
      acc[4 + k] = acc[4 + k] + v * v
    s = (acc[0] + acc[1]) + (acc[2] + acc[3])
    q = (acc[4] + acc[5]) + (acc[6] + acc[7])
    ssum = jnp.sum(s)
    qsum = jnp.sum(q)
    mean = lax.broadcast_in_dim(ssum, (L,), ()) * jnp.float32(INV_H)
    ex2 = lax.broadcast_in_dim(qsum, (L,), ()) * jnp.float32(INV_H)
    var = ex2 - mean * mean
    rstd = _rsqrt_vec(var + jnp.float32(EPS))
    mb = mean * rstd
    for j in range(HV):
      v = inb[r, pl.ds(j * L, L)]
      g = gam[pl.ds(j * L, L)]
      outb[r, pl.ds(j * L, L)] = (v * rstd - mb) * g


def _body(n_tok, ids_hbm, table_hbm, out_hbm,
          idx_v, in0, in1, out0, out1, gam,
          gsem0, gsem1, osem0, osem1):
  rows_per_w = n_tok // NW
  nch = rows_per_w // CHUNK
  inb = (in0, in1)
  outb = (out0, out1)
  gsem = (gsem0, gsem1)
  osem = (osem0, osem1)

  wid = lax.axis_index("s") * NC + lax.axis_index("c")
  row_base = pl.multiple_of(wid * rows_per_w, rows_per_w)

  pltpu.sync_copy(ids_hbm.at[pl.ds(row_base, rows_per_w)], idx_v)

  def do_gather(c, b):
    off = pl.multiple_of(c * CHUNK, CHUNK)
    pltpu.async_copy(table_hbm.at[idx_v.at[pl.ds(off, CHUNK)]], inb[b], gsem[b])

  def wait_gather(b):
    pltpu.make_async_copy(table_hbm.at[pl.ds(0, CHUNK)], inb[b], gsem[b]).wait()

  def do_out(c, b):
    off = pl.multiple_of(row_base + c * CHUNK, CHUNK)
    pltpu.async_copy(outb[b], out_hbm.at[pl.ds(off, CHUNK)], osem[b])

  def wait_out(b):
    pltpu.make_async_copy(outb[b], out_hbm.at[pl.ds(0, CHUNK)], osem[b]).wait()

  do_gather(0, 0)
  do_gather(1, 1)

  def gbody(g, carry):
    for b in (0, 1):
      c = 2 * g + b
      pl.when(g >= 1)(lambda: wait_out(b))
      wait_gather(b)
      _ln_chunk(inb[b], outb[b], gam)
      do_out(c, b)
      pl.when(g < nch // 2 - 1)(lambda: do_gather(c + 2, b))
    return carry

  lax.fori_loop(0, nch // 2, gbody, 0)

  for b in (0, 1):
    wait_out(b)


def _body_with_gamma(n_tok, ids_hbm, table_hbm, gamma_hbm, out_hbm,
                     idx_v, in0, in1, out0, out1, gam,
                     gsem0, gsem1, osem0, osem1):
  pltpu.sync_copy(gamma_hbm, gam)
  _body(n_tok, ids_hbm, table_hbm, out_hbm,
        idx_v, in0, in1, out0, out1, gam,
        gsem0, gsem1, osem0, osem1)


def kernel(input_ids, table, gamma):
  b, s = input_ids.shape
  n_tok = b * s
  rows_per_w = n_tok // NW
  ids_flat = input_ids.reshape((n_tok,))
  mesh = plsc.VectorSubcoreMesh(core_axis_name="c", subcore_axis_name="s")
  run = pl.kernel(
      functools.partial(_body_with_gamma, n_tok),
      out_type=jax.ShapeDtypeStruct((n_tok, H), jnp.float32),
      mesh=mesh,
      compiler_params=pltpu.CompilerParams(needs_layout_passes=False),
      scratch_types=[
          pltpu.VMEM((rows_per_w,), jnp.int32),
          pltpu.VMEM((CHUNK, H), jnp.float32),
          pltpu.VMEM((CHUNK, H), jnp.float32),
          pltpu.VMEM((CHUNK, H), jnp.float32),
          pltpu.VMEM((CHUNK, H), jnp.float32),
          pltpu.VMEM((H,), jnp.float32),
          pltpu.SemaphoreType.DMA,
          pltpu.SemaphoreType.DMA,
          pltpu.SemaphoreType.DMA,
          pltpu.SemaphoreType.DMA,
      ],
  )
  out = run(ids_flat, table, gamma)
  return out.reshape((b, s, H))


# speedup vs baseline: 1.0469x; 1.0469x over previous
import functools

import jax
import jax.numpy as jnp
from jax import lax
from jax.experimental import pallas as pl
from jax.experimental.pallas import tpu as pltpu
from jax.experimental.pallas import tpu_sc as plsc

H = 768
L = 16
NC = 2
NS = 16
NW = NC * NS
HV = H // L
EPS = 1e-5
CHUNK = 32
INV_H = 1.0 / H


def _rsqrt_vec(x):
  i = lax.bitcast_convert_type(x, jnp.int32)
  i = jnp.int32(0x5F3759DF) - lax.shift_right_arithmetic(i, 1)
  y = lax.bitcast_convert_type(i, jnp.float32)
  half_x = x * jnp.float32(0.5)
  for _ in range(2):
    y = y * (jnp.float32(1.5) - half_x * y * y)
  return y


def _ln_chunk(inb, outb, gam, rstd_ref, mb_ref):
  @plsc.parallel_loop(0, CHUNK, unroll=2)
  def stats_body(r):
    s = jnp.zeros((L,), jnp.float32)
    q = jnp.zeros((L,), jnp.float32)
    for j in range(HV):
      v = inb[r, pl.ds(j * L, L)]
      s = s + v
      q = q + v * v
    ssum = jnp.sum(s)
    qsum = jnp.sum(q)
    mean = lax.broadcast_in_dim(ssum, (L,), ()) * jnp.float32(INV_H)
    ex2 = lax.broadcast_in_dim(qsum, (L,), ()) * jnp.float32(INV_H)
    var = ex2 - mean * mean
    rstd = _rsqrt_vec(var + jnp.float32(EPS))
    rstd_ref[r, :] = rstd
    mb_ref[r, :] = mean * rstd

  @plsc.parallel_loop(0, CHUNK, unroll=2)
  def apply_body(r):
    a = rstd_ref[r, :]
    mb = mb_ref[r, :]
    for j in range(HV):
      v = inb[r, pl.ds(j * L, L)]
      g = gam[pl.ds(j * L, L)]
      outb[r, pl.ds(j * L, L)] = (v * a - mb) * g


def _body(n_tok, ids_hbm, table_hbm, out_hbm,
          idx_v, in0, in1, out0, out1, gam, rstd_v, mb_v,
          gsem0, gsem1, osem0, osem1):
  rows_per_w = n_tok // NW
  nch = rows_per_w // CHUNK
  inb = (in0, in1)
  outb = (out0, out1)
  gsem = (gsem0, gsem1)
  osem = (osem0, osem1)

  wid = lax.axis_index("s") * NC + lax.axis_index("c")
  row_base = pl.multiple_of(wid * rows_per_w, rows_per_w)

  pltpu.sync_copy(ids_hbm.at[pl.ds(row_base, rows_per_w)], idx_v)

  def do_gather(c, b):
    off = pl.multiple_of(c * CHUNK, CHUNK)
    pltpu.async_copy(table_hbm.at[idx_v.at[pl.ds(off, CHUNK)]], inb[b], gsem[b])

  def wait_gather(b):
    pltpu.make_async_copy(table_hbm.at[pl.ds(0, CHUNK)], inb[b], gsem[b]).wait()

  def do_out(c, b):
    off = pl.multiple_of(row_base + c * CHUNK, CHUNK)
    pltpu.async_copy(outb[b], out_hbm.at[pl.ds(off, CHUNK)], osem[b])

  def wait_out(b):
    pltpu.make_async_copy(outb[b], out_hbm.at[pl.ds(0, CHUNK)], osem[b]).wait()

  do_gather(0, 0)
  do_gather(1, 1)

  def gbody(g, carry):
    for b in (0, 1):
      c = 2 * g + b
      pl.when(g >= 1)(lambda: wait_out(b))
      wait_gather(b)
      _ln_chunk(inb[b], outb[b], gam, rstd_v, mb_v)
      do_out(c, b)
      pl.when(g < nch // 2 - 1)(lambda: do_gather(c + 2, b))
    return carry

  lax.fori_loop(0, nch // 2, gbody, 0)

  for b in (0, 1):
    wait_out(b)


def _body_with_gamma(n_tok, ids_hbm, table_hbm, gamma_hbm, out_hbm,
                     idx_v, in0, in1, out0, out1, gam, rstd_v, mb_v,
                     gsem0, gsem1, osem0, osem1):
  pltpu.sync_copy(gamma_hbm, gam)
  _body(n_tok, ids_hbm, table_hbm, out_hbm,
        idx_v, in0, in1, out0, out1, gam, rstd_v, mb_v,
        gsem0, gsem1, osem0, osem1)


def kernel(input_ids, table, gamma):
  b, s = input_ids.shape
  n_tok = b * s
  rows_per_w = n_tok // NW
  ids_flat = input_ids.reshape((n_tok,))
  mesh = plsc.VectorSubcoreMesh(core_axis_name="c", subcore_axis_name="s")
  run = pl.kernel(
      functools.partial(_body_with_gamma, n_tok),
      out_type=jax.ShapeDtypeStruct((n_tok, H), jnp.float32),
      mesh=mesh,
      compiler_params=pltpu.CompilerParams(needs_layout_passes=False),
      scratch_types=[
          pltpu.VMEM((rows_per_w,), jnp.int32),
          pltpu.VMEM((CHUNK, H), jnp.float32),
          pltpu.VMEM((CHUNK, H), jnp.float32),
          pltpu.VMEM((CHUNK, H), jnp.float32),
          pltpu.VMEM((CHUNK, H), jnp.float32),
          pltpu.VMEM((H,), jnp.float32),
          pltpu.VMEM((CHUNK, L), jnp.float32),
          pltpu.VMEM((CHUNK, L), jnp.float32),
          pltpu.SemaphoreType.DMA,
          pltpu.SemaphoreType.DMA,
          pltpu.SemaphoreType.DMA,
          pltpu.SemaphoreType.DMA,
      ],
  )
  out = run(ids_flat, table, gamma)
  return out.reshape((b, s, H))
